# initial kernel scaffold (unmeasured)
import functools

import jax
import jax.numpy as jnp
from jax import lax
from jax.experimental import pallas as pl
from jax.experimental.pallas import tpu as pltpu

N_DEV = 4


def _ring_allgather(x_shard):
    m_per, n = x_shard.shape

    def body(x_ref, out_ref, comm_ref, send_sems, recv_sems):
        my = lax.axis_index("i")
        left = (my - 1) % N_DEV
        right = (my + 1) % N_DEV

        barrier_sem = pltpu.get_barrier_semaphore()
        for nbr in [left, right]:
            pl.semaphore_signal(
                barrier_sem, inc=1,
                device_id=(nbr,), device_id_type=pl.DeviceIdType.MESH,
            )
        pl.semaphore_wait(barrier_sem, 2)

        out_ref[pl.ds(my * m_per, m_per), :] = x_ref[:, :]
        comm_ref[0, :, :] = x_ref[:, :]

        for h in range(N_DEV - 1):
            rdma = pltpu.make_async_remote_copy(
                src_ref=comm_ref.at[h],
                dst_ref=comm_ref.at[h + 1],
                send_sem=send_sems.at[h],
                recv_sem=recv_sems.at[h],
                device_id=(right,),
                device_id_type=pl.DeviceIdType.MESH,
            )
            rdma.start()
            rdma.wait()
            origin = (my - h - 1) % N_DEV
            out_ref[pl.ds(origin * m_per, m_per), :] = comm_ref[h + 1, :, :]

    return pl.pallas_call(
        body,
        out_shape=jax.ShapeDtypeStruct((N_DEV * m_per, n), x_shard.dtype),
        in_specs=[pl.BlockSpec(memory_space=pltpu.VMEM)],
        out_specs=pl.BlockSpec(memory_space=pltpu.VMEM),
        scratch_shapes=[
            pltpu.VMEM((N_DEV, m_per, n), x_shard.dtype),
            pltpu.SemaphoreType.DMA((N_DEV - 1,)),
            pltpu.SemaphoreType.DMA((N_DEV - 1,)),
        ],
        compiler_params=pltpu.CompilerParams(collective_id=0),
    )(x_shard)


def _layer_partial(x_full, win, wout, h_chunk=1024):
    m, d = x_full.shape
    h_per = win.shape[1]
    k_steps = h_per // h_chunk

    def body(x_ref, win_ref, wout_ref, out_ref):
        k = pl.program_id(0)
        h = jnp.maximum(
            jnp.dot(x_ref[...], win_ref[...], preferred_element_type=jnp.float32),
            0.0,
        )
        contrib = jnp.dot(h, wout_ref[...], preferred_element_type=jnp.float32)

        @pl.when(k == 0)
        def _():
            out_ref[...] = contrib

        @pl.when(k > 0)
        def _():
            out_ref[...] += contrib

    return pl.pallas_call(
        body,
        grid=(k_steps,),
        in_specs=[
            pl.BlockSpec((m, d), lambda k: (0, 0)),
            pl.BlockSpec((d, h_chunk), lambda k: (0, k)),
            pl.BlockSpec((h_chunk, d), lambda k: (k, 0)),
        ],
        out_specs=pl.BlockSpec((m, d), lambda k: (0, 0)),
        out_shape=jax.ShapeDtypeStruct((m, d), jnp.float32),
        compiler_params=pltpu.CompilerParams(
            dimension_semantics=("arbitrary",),
        ),
    )(x_full, win, wout)


def _ring_allreduce(p):
    m, n = p.shape
    m_per = m // N_DEV

    def body(p_ref, out_ref, comm_ref, rs_send, rs_recv, ag_send, ag_recv):
        my = lax.axis_index("i")
        left = (my - 1) % N_DEV
        right = (my + 1) % N_DEV

        barrier_sem = pltpu.get_barrier_semaphore()
        for nbr in [left, right]:
            pl.semaphore_signal(
                barrier_sem, inc=1,
                device_id=(nbr,), device_id_type=pl.DeviceIdType.MESH,
            )
        pl.semaphore_wait(barrier_sem, 2)

        out_ref[...] = p_ref[...]

        for s in range(N_DEV - 1):
            sc = (my - s) % N_DEV
            rdma = pltpu.make_async_remote_copy(
                src_ref=out_ref.at[pl.ds(sc * m_per, m_per), :],
                dst_ref=comm_ref.at[s],
                send_sem=rs_send.at[s],
                recv_sem=rs_recv.at[s],
                device_id=(right,),
                device_id_type=pl.DeviceIdType.MESH,
            )
            rdma.start()
            rdma.wait()
            rc = (my - s - 1) % N_DEV
            out_ref[pl.ds(rc * m_per, m_per), :] += comm_ref[s, :, :]

        for t in range(N_DEV - 1):
            gc = (my + 1 - t) % N_DEV
            rdma = pltpu.make_async_remote_copy(
                src_ref=out_ref.at[pl.ds(gc * m_per, m_per), :],
                dst_ref=out_ref.at[pl.ds(gc * m_per, m_per), :],
                send_sem=ag_send.at[t],
                recv_sem=ag_recv.at[t],
                device_id=(right,),
                device_id_type=pl.DeviceIdType.MESH,
            )
            rdma.start()
            rdma.wait()

    return pl.pallas_call(
        body,
        out_shape=jax.ShapeDtypeStruct((m, n), jnp.float32),
        in_specs=[pl.BlockSpec(memory_space=pltpu.VMEM)],
        out_specs=pl.BlockSpec(memory_space=pltpu.VMEM),
        scratch_shapes=[
            pltpu.VMEM((N_DEV - 1, m_per, n), jnp.float32),
            pltpu.SemaphoreType.DMA((N_DEV - 1,)),
            pltpu.SemaphoreType.DMA((N_DEV - 1,)),
            pltpu.SemaphoreType.DMA((N_DEV - 1,)),
            pltpu.SemaphoreType.DMA((N_DEV - 1,)),
        ],
        compiler_params=pltpu.CompilerParams(collective_id=1),
    )(p)


def kernel(x, Win0, Wout0, Win1, Wout1, Win2, Wout2):
    X = _ring_allgather(x)
    for win, wout in ((Win0, Wout0), (Win1, Wout1), (Win2, Wout2)):
        p = _layer_partial(X, win, wout)
        X = _ring_allreduce(p)
    return X


# baseline (device time: 254006 ns/iter reference)
import functools

import jax
import jax.numpy as jnp
from jax import lax
from jax.experimental import pallas as pl
from jax.experimental.pallas import tpu as pltpu

N_DEV = 4


def _ring_allgather(x_shard):
    m_per, n = x_shard.shape

    def body(x_ref, out_ref, comm_ref, send_sems, recv_sems):
        my = lax.axis_index("i")
        left = (my - 1) % N_DEV
        right = (my + 1) % N_DEV

        barrier_sem = pltpu.get_barrier_semaphore()
        for nbr in [left, right]:
            pl.semaphore_signal(
                barrier_sem, inc=1,
                device_id=(nbr,), device_id_type=pl.DeviceIdType.MESH,
            )
        pl.semaphore_wait(barrier_sem, 2)

        out_ref[pl.ds(my * m_per, m_per), :] = x_ref[:, :]
        comm_ref[0, :, :] = x_ref[:, :]

        for h in range(N_DEV - 1):
            rdma = pltpu.make_async_remote_copy(
                src_ref=comm_ref.at[h],
                dst_ref=comm_ref.at[h + 1],
                send_sem=send_sems.at[h],
                recv_sem=recv_sems.at[h],
                device_id=(right,),
                device_id_type=pl.DeviceIdType.MESH,
            )
            rdma.start()
            rdma.wait()
            origin = (my - h - 1) % N_DEV
            out_ref[pl.ds(origin * m_per, m_per), :] = comm_ref[h + 1, :, :]

    return pl.pallas_call(
        body,
        out_shape=jax.ShapeDtypeStruct((N_DEV * m_per, n), x_shard.dtype),
        in_specs=[pl.BlockSpec(memory_space=pltpu.VMEM)],
        out_specs=pl.BlockSpec(memory_space=pltpu.VMEM),
        scratch_shapes=[
            pltpu.VMEM((N_DEV, m_per, n), x_shard.dtype),
            pltpu.SemaphoreType.DMA((N_DEV - 1,)),
            pltpu.SemaphoreType.DMA((N_DEV - 1,)),
        ],
        compiler_params=pltpu.CompilerParams(collective_id=0),
    )(x_shard)


def _layer_partial(x_full, win, wout, h_chunk=1024):
    m, d = x_full.shape
    h_per = win.shape[1]
    k_steps = h_per // h_chunk

    def body(x_ref, win_ref, wout_ref, out_ref):
        k = pl.program_id(0)
        h = jnp.maximum(
            jnp.dot(x_ref[...], win_ref[...], preferred_element_type=jnp.float32),
            0.0,
        )
        contrib = jnp.dot(h, wout_ref[...], preferred_element_type=jnp.float32)

        @pl.when(k == 0)
        def _():
            out_ref[...] = contrib

        @pl.when(k > 0)
        def _():
            out_ref[...] += contrib

    return pl.pallas_call(
        body,
        grid=(k_steps,),
        in_specs=[
            pl.BlockSpec((m, d), lambda k: (0, 0)),
            pl.BlockSpec((d, h_chunk), lambda k: (0, k)),
            pl.BlockSpec((h_chunk, d), lambda k: (k, 0)),
        ],
        out_specs=pl.BlockSpec((m, d), lambda k: (0, 0)),
        out_shape=jax.ShapeDtypeStruct((m, d), jnp.float32),
        compiler_params=pltpu.CompilerParams(
            dimension_semantics=("arbitrary",),
            vmem_limit_bytes=60 * 1024 * 1024,
        ),
    )(x_full, win, wout)


def _ring_allreduce(p):
    m, n = p.shape
    m_per = m // N_DEV

    def body(p_ref, out_ref, comm_ref, rs_send, rs_recv, ag_send, ag_recv):
        my = lax.axis_index("i")
        left = (my - 1) % N_DEV
        right = (my + 1) % N_DEV

        barrier_sem = pltpu.get_barrier_semaphore()
        for nbr in [left, right]:
            pl.semaphore_signal(
                barrier_sem, inc=1,
                device_id=(nbr,), device_id_type=pl.DeviceIdType.MESH,
            )
        pl.semaphore_wait(barrier_sem, 2)

        out_ref[...] = p_ref[...]

        for s in range(N_DEV - 1):
            sc = (my - s) % N_DEV
            rdma = pltpu.make_async_remote_copy(
                src_ref=out_ref.at[pl.ds(sc * m_per, m_per), :],
                dst_ref=comm_ref.at[s],
                send_sem=rs_send.at[s],
                recv_sem=rs_recv.at[s],
                device_id=(right,),
                device_id_type=pl.DeviceIdType.MESH,
            )
            rdma.start()
            rdma.wait()
            rc = (my - s - 1) % N_DEV
            out_ref[pl.ds(rc * m_per, m_per), :] += comm_ref[s, :, :]

        for t in range(N_DEV - 1):
            gc = (my + 1 - t) % N_DEV
            rdma = pltpu.make_async_remote_copy(
                src_ref=out_ref.at[pl.ds(gc * m_per, m_per), :],
                dst_ref=out_ref.at[pl.ds(gc * m_per, m_per), :],
                send_sem=ag_send.at[t],
                recv_sem=ag_recv.at[t],
                device_id=(right,),
                device_id_type=pl.DeviceIdType.MESH,
            )
            rdma.start()
            rdma.wait()

    return pl.pallas_call(
        body,
        out_shape=jax.ShapeDtypeStruct((m, n), jnp.float32),
        in_specs=[pl.BlockSpec(memory_space=pltpu.VMEM)],
        out_specs=pl.BlockSpec(memory_space=pltpu.VMEM),
        scratch_shapes=[
            pltpu.VMEM((N_DEV - 1, m_per, n), jnp.float32),
            pltpu.SemaphoreType.DMA((N_DEV - 1,)),
            pltpu.SemaphoreType.DMA((N_DEV - 1,)),
            pltpu.SemaphoreType.DMA((N_DEV - 1,)),
            pltpu.SemaphoreType.DMA((N_DEV - 1,)),
        ],
        compiler_params=pltpu.CompilerParams(collective_id=1),
    )(p)


def kernel(x, Win0, Wout0, Win1, Wout1, Win2, Wout2):
    X = _ring_allgather(x)
    for win, wout in ((Win0, Wout0), (Win1, Wout1), (Win2, Wout2)):
        p = _layer_partial(X, win, wout)
        X = _ring_allreduce(p)
    return X


# device time: 195352 ns/iter; 1.3002x vs baseline; 1.3002x over previous
import functools

import jax
import jax.numpy as jnp
from jax import lax
from jax.experimental import pallas as pl
from jax.experimental.pallas import tpu as pltpu

N_DEV = 4


def _ring_allgather(x_shard):
    m_per, n = x_shard.shape

    def body(x_ref, out_ref, comm_ref, send_sems, recv_sems):
        my = lax.axis_index("i")
        left = (my - 1) % N_DEV
        right = (my + 1) % N_DEV

        barrier_sem = pltpu.get_barrier_semaphore()
        for nbr in [left, right]:
            pl.semaphore_signal(
                barrier_sem, inc=1,
                device_id=(nbr,), device_id_type=pl.DeviceIdType.MESH,
            )
        pl.semaphore_wait(barrier_sem, 2)

        out_ref[pl.ds(my * m_per, m_per), :] = x_ref[:, :]
        comm_ref[0, :, :] = x_ref[:, :].astype(jnp.bfloat16)

        for h in range(N_DEV - 1):
            rdma = pltpu.make_async_remote_copy(
                src_ref=comm_ref.at[h],
                dst_ref=comm_ref.at[h + 1],
                send_sem=send_sems.at[h],
                recv_sem=recv_sems.at[h],
                device_id=(right,),
                device_id_type=pl.DeviceIdType.MESH,
            )
            rdma.start()
            rdma.wait()
            origin = (my - h - 1) % N_DEV
            out_ref[pl.ds(origin * m_per, m_per), :] = comm_ref[h + 1, :, :].astype(
                jnp.float32
            )

    return pl.pallas_call(
        body,
        out_shape=jax.ShapeDtypeStruct((N_DEV * m_per, n), x_shard.dtype),
        in_specs=[pl.BlockSpec(memory_space=pltpu.VMEM)],
        out_specs=pl.BlockSpec(memory_space=pltpu.VMEM),
        scratch_shapes=[
            pltpu.VMEM((N_DEV, m_per, n), jnp.bfloat16),
            pltpu.SemaphoreType.DMA((N_DEV - 1,)),
            pltpu.SemaphoreType.DMA((N_DEV - 1,)),
        ],
        compiler_params=pltpu.CompilerParams(collective_id=0),
    )(x_shard)


def _layer_partial(x_full, win, wout, h_chunk=1024):
    m, d = x_full.shape
    h_per = win.shape[1]
    k_steps = h_per // h_chunk

    def body(x_ref, win_ref, wout_ref, out_ref):
        k = pl.program_id(0)
        h = jnp.maximum(
            jnp.dot(x_ref[...], win_ref[...], preferred_element_type=jnp.float32),
            0.0,
        )
        contrib = jnp.dot(h, wout_ref[...], preferred_element_type=jnp.float32)

        @pl.when(k == 0)
        def _():
            out_ref[...] = contrib

        @pl.when(k > 0)
        def _():
            out_ref[...] += contrib

    return pl.pallas_call(
        body,
        grid=(k_steps,),
        in_specs=[
            pl.BlockSpec((m, d), lambda k: (0, 0)),
            pl.BlockSpec((d, h_chunk), lambda k: (0, k)),
            pl.BlockSpec((h_chunk, d), lambda k: (k, 0)),
        ],
        out_specs=pl.BlockSpec((m, d), lambda k: (0, 0)),
        out_shape=jax.ShapeDtypeStruct((m, d), jnp.float32),
        compiler_params=pltpu.CompilerParams(
            dimension_semantics=("arbitrary",),
            vmem_limit_bytes=60 * 1024 * 1024,
        ),
    )(x_full, win, wout)


def _ring_allreduce(p):
    m, n = p.shape
    m_per = m // N_DEV

    def body(
        p_ref, out_ref, rs_stage, rs_comm, ag_stage, ag_comm,
        rs_send, rs_recv, ag_send, ag_recv,
    ):
        my = lax.axis_index("i")
        left = (my - 1) % N_DEV
        right = (my + 1) % N_DEV

        barrier_sem = pltpu.get_barrier_semaphore()
        for nbr in [left, right]:
            pl.semaphore_signal(
                barrier_sem, inc=1,
                device_id=(nbr,), device_id_type=pl.DeviceIdType.MESH,
            )
        pl.semaphore_wait(barrier_sem, 2)

        out_ref[...] = p_ref[...]

        for s in range(N_DEV - 1):
            sc = (my - s) % N_DEV
            rs_stage[s, :, :] = out_ref[pl.ds(sc * m_per, m_per), :].astype(
                jnp.bfloat16
            )
            rdma = pltpu.make_async_remote_copy(
                src_ref=rs_stage.at[s],
                dst_ref=rs_comm.at[s],
                send_sem=rs_send.at[s],
                recv_sem=rs_recv.at[s],
                device_id=(right,),
                device_id_type=pl.DeviceIdType.MESH,
            )
            rdma.start()
            rdma.wait()
            rc = (my - s - 1) % N_DEV
            out_ref[pl.ds(rc * m_per, m_per), :] += rs_comm[s, :, :].astype(
                jnp.float32
            )

        for t in range(N_DEV - 1):
            if t == 0:
                own = (my + 1) % N_DEV
                ag_stage[0, :, :] = out_ref[pl.ds(own * m_per, m_per), :].astype(
                    jnp.bfloat16
                )
                src = ag_stage.at[0]
            else:
                src = ag_comm.at[t - 1]
            rdma = pltpu.make_async_remote_copy(
                src_ref=src,
                dst_ref=ag_comm.at[t],
                send_sem=ag_send.at[t],
                recv_sem=ag_recv.at[t],
                device_id=(right,),
                device_id_type=pl.DeviceIdType.MESH,
            )
            rdma.start()
            rdma.wait()
            rc = (my - t) % N_DEV
            out_ref[pl.ds(rc * m_per, m_per), :] = ag_comm[t, :, :].astype(
                jnp.float32
            )

    return pl.pallas_call(
        body,
        out_shape=jax.ShapeDtypeStruct((m, n), jnp.float32),
        in_specs=[pl.BlockSpec(memory_space=pltpu.VMEM)],
        out_specs=pl.BlockSpec(memory_space=pltpu.VMEM),
        scratch_shapes=[
            pltpu.VMEM((N_DEV - 1, m_per, n), jnp.bfloat16),
            pltpu.VMEM((N_DEV - 1, m_per, n), jnp.bfloat16),
            pltpu.VMEM((1, m_per, n), jnp.bfloat16),
            pltpu.VMEM((N_DEV - 1, m_per, n), jnp.bfloat16),
            pltpu.SemaphoreType.DMA((N_DEV - 1,)),
            pltpu.SemaphoreType.DMA((N_DEV - 1,)),
            pltpu.SemaphoreType.DMA((N_DEV - 1,)),
            pltpu.SemaphoreType.DMA((N_DEV - 1,)),
        ],
        compiler_params=pltpu.CompilerParams(collective_id=1),
    )(p)


def kernel(x, Win0, Wout0, Win1, Wout1, Win2, Wout2):
    X = _ring_allgather(x)
    for win, wout in ((Win0, Wout0), (Win1, Wout1), (Win2, Wout2)):
        p = _layer_partial(X, win, wout)
        X = _ring_allreduce(p)
    return X


# device time: 141749 ns/iter; 1.7919x vs baseline; 1.3782x over previous
import jax
import jax.numpy as jnp
from jax import lax
from jax.experimental import pallas as pl
from jax.experimental.pallas import tpu as pltpu

N_DEV = 4
M_PER = 64
D = 2048
H_PER = 4096
CH = 512
KS = H_PER // CH
N_LAYERS = 3
NCH = N_LAYERS * KS
SLOTS = 5


def kernel(x, Win0, Wout0, Win1, Wout1, Win2, Wout2):
    def body(
        x_ref, win0, wout0, win1, wout1, win2, wout2, out_ref,
        X, ACC, WINB, WOUTB, AGX, RSS, RSC, AGS, AGC,
        agx_ssem, agx_rsem, rs_ssem, rs_rsem, ag_ssem, ag_rsem,
        wi_sem, wo_sem,
    ):
        my = lax.axis_index("i")
        left = (my - 1) % N_DEV
        right = (my + 1) % N_DEV
        wins = [win0, win1, win2]
        wouts = [wout0, wout1, wout2]

        def w_copies(g):
            l, k = divmod(g, KS)
            s = g % SLOTS
            return (
                pltpu.make_async_copy(
                    wins[l].at[:, k * CH:(k + 1) * CH], WINB.at[s], wi_sem.at[s]
                ),
                pltpu.make_async_copy(
                    wouts[l].at[k * CH:(k + 1) * CH, :], WOUTB.at[s], wo_sem.at[s]
                ),
            )

        for g in range(SLOTS):
            a, b = w_copies(g)
            a.start()
            b.start()

        barrier_sem = pltpu.get_barrier_semaphore()
        for nbr in [left, right]:
            pl.semaphore_signal(
                barrier_sem, inc=1,
                device_id=(nbr,), device_id_type=pl.DeviceIdType.MESH,
            )
        pl.semaphore_wait(barrier_sem, 2)

        X[pl.ds(my * M_PER, M_PER), :] = x_ref[...]
        AGX[0, :, :] = x_ref[...].astype(jnp.bfloat16)
        for h in range(N_DEV - 1):
            rdma = pltpu.make_async_remote_copy(
                src_ref=AGX.at[h],
                dst_ref=AGX.at[h + 1],
                send_sem=agx_ssem.at[h],
                recv_sem=agx_rsem.at[h],
                device_id=(right,),
                device_id_type=pl.DeviceIdType.MESH,
            )
            rdma.start()
            rdma.wait()
            origin = (my - h - 1) % N_DEV
            X[pl.ds(origin * M_PER, M_PER), :] = AGX[h + 1, :, :].astype(jnp.float32)

        for l in range(N_LAYERS):
            for k in range(KS):
                g = l * KS + k
                a, b = w_copies(g)
                a.wait()
                b.wait()
                s = g % SLOTS
                h = jnp.maximum(
                    jnp.dot(X[...], WINB[s], preferred_element_type=jnp.float32),
                    0.0,
                )
                contrib = jnp.dot(h, WOUTB[s], preferred_element_type=jnp.float32)
                if k == 0:
                    ACC[...] = contrib
                else:
                    ACC[...] += contrib
                if g + SLOTS < NCH:
                    a2, b2 = w_copies(g + SLOTS)
                    a2.start()
                    b2.start()

            for s3 in range(N_DEV - 1):
                sc = (my - s3) % N_DEV
                RSS[s3, :, :] = ACC[pl.ds(sc * M_PER, M_PER), :].astype(jnp.bfloat16)
                rdma = pltpu.make_async_remote_copy(
                    src_ref=RSS.at[s3],
                    dst_ref=RSC.at[s3],
                    send_sem=rs_ssem.at[s3],
                    recv_sem=rs_rsem.at[s3],
                    device_id=(right,),
                    device_id_type=pl.DeviceIdType.MESH,
                )
                rdma.start()
                rdma.wait()
                rc = (my - s3 - 1) % N_DEV
                ACC[pl.ds(rc * M_PER, M_PER), :] += RSC[s3, :, :].astype(jnp.float32)

            own = (my + 1) % N_DEV
            dest = X if l < N_LAYERS - 1 else out_ref
            dest[pl.ds(own * M_PER, M_PER), :] = ACC[pl.ds(own * M_PER, M_PER), :]
            for t in range(N_DEV - 1):
                if t == 0:
                    AGS[0, :, :] = ACC[pl.ds(own * M_PER, M_PER), :].astype(
                        jnp.bfloat16
                    )
                    src = AGS.at[0]
                else:
                    src = AGC.at[t - 1]
                rdma = pltpu.make_async_remote_copy(
                    src_ref=src,
                    dst_ref=AGC.at[t],
                    send_sem=ag_ssem.at[t],
                    recv_sem=ag_rsem.at[t],
                    device_id=(right,),
                    device_id_type=pl.DeviceIdType.MESH,
                )
                rdma.start()
                rdma.wait()
                rc = (my - t) % N_DEV
                dest[pl.ds(rc * M_PER, M_PER), :] = AGC[t, :, :].astype(jnp.float32)

    return pl.pallas_call(
        body,
        out_shape=jax.ShapeDtypeStruct((N_DEV * M_PER, D), jnp.float32),
        in_specs=[
            pl.BlockSpec(memory_space=pltpu.VMEM),
            pl.BlockSpec(memory_space=pltpu.MemorySpace.HBM),
            pl.BlockSpec(memory_space=pltpu.MemorySpace.HBM),
            pl.BlockSpec(memory_space=pltpu.MemorySpace.HBM),
            pl.BlockSpec(memory_space=pltpu.MemorySpace.HBM),
            pl.BlockSpec(memory_space=pltpu.MemorySpace.HBM),
            pl.BlockSpec(memory_space=pltpu.MemorySpace.HBM),
        ],
        out_specs=pl.BlockSpec(memory_space=pltpu.VMEM),
        scratch_shapes=[
            pltpu.VMEM((N_DEV * M_PER, D), jnp.float32),
            pltpu.VMEM((N_DEV * M_PER, D), jnp.float32),
            pltpu.VMEM((SLOTS, D, CH), jnp.float32),
            pltpu.VMEM((SLOTS, CH, D), jnp.float32),
            pltpu.VMEM((N_DEV, M_PER, D), jnp.bfloat16),
            pltpu.VMEM((N_DEV - 1, M_PER, D), jnp.bfloat16),
            pltpu.VMEM((N_DEV - 1, M_PER, D), jnp.bfloat16),
            pltpu.VMEM((1, M_PER, D), jnp.bfloat16),
            pltpu.VMEM((N_DEV - 1, M_PER, D), jnp.bfloat16),
            pltpu.SemaphoreType.DMA((N_DEV - 1,)),
            pltpu.SemaphoreType.DMA((N_DEV - 1,)),
            pltpu.SemaphoreType.DMA((N_DEV - 1,)),
            pltpu.SemaphoreType.DMA((N_DEV - 1,)),
            pltpu.SemaphoreType.DMA((N_DEV - 1,)),
            pltpu.SemaphoreType.DMA((N_DEV - 1,)),
            pltpu.SemaphoreType.DMA((SLOTS,)),
            pltpu.SemaphoreType.DMA((SLOTS,)),
        ],
        compiler_params=pltpu.CompilerParams(
            collective_id=0,
            vmem_limit_bytes=60 * 1024 * 1024,
        ),
    )(x, Win0, Wout0, Win1, Wout1, Win2, Wout2)


# device time: 114267 ns/iter; 2.2229x vs baseline; 1.2405x over previous
import jax
import jax.numpy as jnp
from jax import lax
from jax.experimental import pallas as pl
from jax.experimental.pallas import tpu as pltpu

N_DEV = 4
M_PER = 64
D = 2048
DH = D // 2
H_PER = 4096
CH = 512
KS = H_PER // CH
N_LAYERS = 3
NCH = N_LAYERS * KS
SLOTS = 5

F32 = jnp.float32
BF16 = jnp.bfloat16


def kernel(x, Win0, Wout0, Win1, Wout1, Win2, Wout2):
    def body(
        x_ref, win0, wout0, win1, wout1, win2, wout2, out_ref,
        X, ACC, WINB, WOUTB,
        AGXA, AGXB, RSSA, RSCA, RSSB, RSCB, AGSA, AGCA, AGSB, AGCB,
        agxa_s, agxa_r, agxb_s, agxb_r,
        rsa_s, rsa_r, rsb_s, rsb_r,
        aga_s, aga_r, agb_s, agb_r,
        wi_sem, wo_sem,
    ):
        my = lax.axis_index("i")
        left = (my - 1) % N_DEV
        right = (my + 1) % N_DEV
        wins = [win0, win1, win2]
        wouts = [wout0, wout1, wout2]

        def w_copies(g):
            l, k = divmod(g, KS)
            s = g % SLOTS
            return (
                pltpu.make_async_copy(
                    wins[l].at[:, k * CH:(k + 1) * CH], WINB.at[s], wi_sem.at[s]
                ),
                pltpu.make_async_copy(
                    wouts[l].at[k * CH:(k + 1) * CH, :], WOUTB.at[s], wo_sem.at[s]
                ),
            )

        def rdma(src, dst, ssem, rsem, target):
            return pltpu.make_async_remote_copy(
                src_ref=src, dst_ref=dst, send_sem=ssem, recv_sem=rsem,
                device_id=(target,), device_id_type=pl.DeviceIdType.MESH,
            )

        for g in range(SLOTS):
            a, b = w_copies(g)
            a.start()
            b.start()

        barrier_sem = pltpu.get_barrier_semaphore()
        for nbr in [left, right]:
            pl.semaphore_signal(
                barrier_sem, inc=1,
                device_id=(nbr,), device_id_type=pl.DeviceIdType.MESH,
            )
        pl.semaphore_wait(barrier_sem, 2)

        X[pl.ds(my * M_PER, M_PER), :] = x_ref[...]
        AGXA[0, :, :] = x_ref[:, :DH].astype(BF16)
        AGXB[0, :, :] = x_ref[:, DH:].astype(BF16)
        sends = []
        for h in range(N_DEV - 1):
            ra = rdma(AGXA.at[h], AGXA.at[h + 1], agxa_s.at[h], agxa_r.at[h], right)
            rb = rdma(AGXB.at[h], AGXB.at[h + 1], agxb_s.at[h], agxb_r.at[h], left)
            ra.start()
            rb.start()
            ra.wait_recv()
            oa = (my - h - 1) % N_DEV
            X[pl.ds(oa * M_PER, M_PER), :DH] = AGXA[h + 1, :, :].astype(F32)
            rb.wait_recv()
            ob = (my + h + 1) % N_DEV
            X[pl.ds(ob * M_PER, M_PER), DH:] = AGXB[h + 1, :, :].astype(F32)
            sends += [ra, rb]
        for r in sends:
            r.wait_send()

        for l in range(N_LAYERS):
            for k in range(KS):
                g = l * KS + k
                a, b = w_copies(g)
                a.wait()
                b.wait()
                s = g % SLOTS
                h = jnp.maximum(
                    jnp.dot(X[...], WINB[s], preferred_element_type=F32), 0.0
                )
                contrib = jnp.dot(h, WOUTB[s], preferred_element_type=F32)
                if k == 0:
                    ACC[...] = contrib
                else:
                    ACC[...] += contrib
                if g + SLOTS < NCH:
                    a2, b2 = w_copies(g + SLOTS)
                    a2.start()
                    b2.start()

            sends = []
            for s3 in range(N_DEV - 1):
                sca = (my - s3) % N_DEV
                scb = (my + s3) % N_DEV
                RSSA[s3, :, :] = ACC[pl.ds(sca * M_PER, M_PER), :DH].astype(BF16)
                RSSB[s3, :, :] = ACC[pl.ds(scb * M_PER, M_PER), DH:].astype(BF16)
                ra = rdma(RSSA.at[s3], RSCA.at[s3], rsa_s.at[s3], rsa_r.at[s3], right)
                rb = rdma(RSSB.at[s3], RSCB.at[s3], rsb_s.at[s3], rsb_r.at[s3], left)
                ra.start()
                rb.start()
                ra.wait_recv()
                rca = (my - s3 - 1) % N_DEV
                ACC[pl.ds(rca * M_PER, M_PER), :DH] += RSCA[s3, :, :].astype(F32)
                rb.wait_recv()
                rcb = (my + s3 + 1) % N_DEV
                ACC[pl.ds(rcb * M_PER, M_PER), DH:] += RSCB[s3, :, :].astype(F32)
                sends += [ra, rb]
            for r in sends:
                r.wait_send()

            own_a = (my + 1) % N_DEV
            own_b = (my - 1) % N_DEV
            dest = X if l < N_LAYERS - 1 else out_ref
            dest[pl.ds(own_a * M_PER, M_PER), :DH] = ACC[
                pl.ds(own_a * M_PER, M_PER), :DH
            ]
            dest[pl.ds(own_b * M_PER, M_PER), DH:] = ACC[
                pl.ds(own_b * M_PER, M_PER), DH:
            ]
            sends = []
            for t in range(N_DEV - 1):
                if t == 0:
                    AGSA[0, :, :] = ACC[pl.ds(own_a * M_PER, M_PER), :DH].astype(BF16)
                    AGSB[0, :, :] = ACC[pl.ds(own_b * M_PER, M_PER), DH:].astype(BF16)
                    src_a, src_b = AGSA.at[0], AGSB.at[0]
                else:
                    src_a, src_b = AGCA.at[t - 1], AGCB.at[t - 1]
                ra = rdma(src_a, AGCA.at[t], aga_s.at[t], aga_r.at[t], right)
                rb = rdma(src_b, AGCB.at[t], agb_s.at[t], agb_r.at[t], left)
                ra.start()
                rb.start()
                ra.wait_recv()
                rca = (my - t) % N_DEV
                dest[pl.ds(rca * M_PER, M_PER), :DH] = AGCA[t, :, :].astype(F32)
                rb.wait_recv()
                rcb = (my + t) % N_DEV
                dest[pl.ds(rcb * M_PER, M_PER), DH:] = AGCB[t, :, :].astype(F32)
                sends += [ra, rb]
            for r in sends:
                r.wait_send()

    return pl.pallas_call(
        body,
        out_shape=jax.ShapeDtypeStruct((N_DEV * M_PER, D), F32),
        in_specs=[pl.BlockSpec(memory_space=pltpu.VMEM)]
        + [pl.BlockSpec(memory_space=pltpu.MemorySpace.HBM)] * 6,
        out_specs=pl.BlockSpec(memory_space=pltpu.VMEM),
        scratch_shapes=[
            pltpu.VMEM((N_DEV * M_PER, D), F32),
            pltpu.VMEM((N_DEV * M_PER, D), F32),
            pltpu.VMEM((SLOTS, D, CH), F32),
            pltpu.VMEM((SLOTS, CH, D), F32),
            pltpu.VMEM((N_DEV, M_PER, DH), BF16),
            pltpu.VMEM((N_DEV, M_PER, DH), BF16),
            pltpu.VMEM((N_DEV - 1, M_PER, DH), BF16),
            pltpu.VMEM((N_DEV - 1, M_PER, DH), BF16),
            pltpu.VMEM((N_DEV - 1, M_PER, DH), BF16),
            pltpu.VMEM((N_DEV - 1, M_PER, DH), BF16),
            pltpu.VMEM((1, M_PER, DH), BF16),
            pltpu.VMEM((N_DEV - 1, M_PER, DH), BF16),
            pltpu.VMEM((1, M_PER, DH), BF16),
            pltpu.VMEM((N_DEV - 1, M_PER, DH), BF16),
        ]
        + [pltpu.SemaphoreType.DMA((N_DEV - 1,))] * 12
        + [
            pltpu.SemaphoreType.DMA((SLOTS,)),
            pltpu.SemaphoreType.DMA((SLOTS,)),
        ],
        compiler_params=pltpu.CompilerParams(
            collective_id=0,
            vmem_limit_bytes=60 * 1024 * 1024,
        ),
    )(x, Win0, Wout0, Win1, Wout1, Win2, Wout2)


# device time: 103663 ns/iter; 2.4503x vs baseline; 1.1023x over previous
import jax
import jax.numpy as jnp
from jax import lax
from jax.experimental import pallas as pl
from jax.experimental.pallas import tpu as pltpu

N_DEV = 4
M_PER = 64
D = 2048
DH = D // 2
H_PER = 4096
CH = 512
KS = H_PER // CH
N_LAYERS = 3
NCH = N_LAYERS * KS
SLOTS = 5

F32 = jnp.float32
BF16 = jnp.bfloat16


def kernel(x, Win0, Wout0, Win1, Wout1, Win2, Wout2):
    def body(
        x_ref, win0, wout0, win1, wout1, win2, wout2, out_ref,
        X, ACC, WINB, WOUTB,
        AGXA, AGXB, DRSO, DRSI, DOA, DOB, DIA, DIB,
        BO, BOA, BOB, BIN, BIA, BIB,
        agxa_s, agxa_r, agxb_s, agxb_r,
        drs_s, drs_r, bag_s, bag_r,
        wi_sem, wo_sem,
    ):
        my = lax.axis_index("i")
        left = (my - 1) % N_DEV
        right = (my + 1) % N_DEV
        diag = (my + 2) % N_DEV
        wins = [win0, win1, win2]
        wouts = [wout0, wout1, wout2]

        def w_copies(g):
            l, k = divmod(g, KS)
            s = g % SLOTS
            return (
                pltpu.make_async_copy(
                    wins[l].at[:, k * CH:(k + 1) * CH], WINB.at[s], wi_sem.at[s]
                ),
                pltpu.make_async_copy(
                    wouts[l].at[k * CH:(k + 1) * CH, :], WOUTB.at[s], wo_sem.at[s]
                ),
            )

        def rdma(src, dst, ssem, rsem, target):
            return pltpu.make_async_remote_copy(
                src_ref=src, dst_ref=dst, send_sem=ssem, recv_sem=rsem,
                device_id=(target,), device_id_type=pl.DeviceIdType.MESH,
            )

        for g in range(SLOTS):
            a, b = w_copies(g)
            a.start()
            b.start()

        barrier_sem = pltpu.get_barrier_semaphore()
        for nbr in [left, right, diag]:
            pl.semaphore_signal(
                barrier_sem, inc=1,
                device_id=(nbr,), device_id_type=pl.DeviceIdType.MESH,
            )
        pl.semaphore_wait(barrier_sem, 3)

        X[pl.ds(my * M_PER, M_PER), :] = x_ref[...]
        AGXA[0, :, :] = x_ref[:, :DH].astype(BF16)
        AGXB[0, :, :] = x_ref[:, DH:].astype(BF16)
        sends = []
        for h in range(N_DEV - 1):
            ra = rdma(AGXA.at[h], AGXA.at[h + 1], agxa_s.at[h], agxa_r.at[h], right)
            rb = rdma(AGXB.at[h], AGXB.at[h + 1], agxb_s.at[h], agxb_r.at[h], left)
            ra.start()
            rb.start()
            ra.wait_recv()
            oa = (my - h - 1) % N_DEV
            X[pl.ds(oa * M_PER, M_PER), :DH] = AGXA[h + 1, :, :].astype(F32)
            rb.wait_recv()
            ob = (my + h + 1) % N_DEV
            X[pl.ds(ob * M_PER, M_PER), DH:] = AGXB[h + 1, :, :].astype(F32)
            sends += [ra, rb]
        for r in sends:
            r.wait_send()

        for l in range(N_LAYERS):
            for k in range(KS):
                g = l * KS + k
                a, b = w_copies(g)
                a.wait()
                b.wait()
                s = g % SLOTS
                h = jnp.maximum(
                    jnp.dot(X[...], WINB[s], preferred_element_type=F32), 0.0
                )
                contrib = jnp.dot(h, WOUTB[s], preferred_element_type=F32)
                if k == 0:
                    ACC[...] = contrib
                else:
                    ACC[...] += contrib
                if g + SLOTS < NCH:
                    a2, b2 = w_copies(g + SLOTS)
                    a2.start()
                    b2.start()

            DRSO[0, :, :] = ACC[pl.ds(right * M_PER, M_PER), :].astype(BF16)
            DRSO[1, :, :] = ACC[pl.ds(left * M_PER, M_PER), :].astype(BF16)
            DOA[0, :, :] = ACC[pl.ds(diag * M_PER, M_PER), :DH].astype(BF16)
            DOB[0, :, :] = ACC[pl.ds(diag * M_PER, M_PER), DH:].astype(BF16)
            r0 = rdma(DRSO.at[0], DRSI.at[0], drs_s.at[0], drs_r.at[0], right)
            r1 = rdma(DRSO.at[1], DRSI.at[1], drs_s.at[1], drs_r.at[1], left)
            r2 = rdma(DOA.at[0], DIA.at[0], drs_s.at[2], drs_r.at[2], diag)
            r3 = rdma(DOB.at[0], DIB.at[0], drs_s.at[3], drs_r.at[3], diag)
            r0.start()
            r1.start()
            r2.start()
            r3.start()
            r0.wait_recv()
            r1.wait_recv()
            myrows = pl.ds(my * M_PER, M_PER)
            ACC[myrows, :] += DRSI[0, :, :].astype(F32) + DRSI[1, :, :].astype(F32)
            r2.wait_recv()
            ACC[myrows, :DH] += DIA[0, :, :].astype(F32)
            r3.wait_recv()
            ACC[myrows, DH:] += DIB[0, :, :].astype(F32)
            for r in (r0, r1, r2, r3):
                r.wait_send()

            dest = X if l < N_LAYERS - 1 else out_ref
            dest[myrows, :] = ACC[myrows, :]
            BO[0, :, :] = ACC[myrows, :].astype(BF16)
            BOA[0, :, :] = ACC[myrows, :DH].astype(BF16)
            BOB[0, :, :] = ACC[myrows, DH:].astype(BF16)
            b0 = rdma(BO.at[0], BIN.at[0], bag_s.at[0], bag_r.at[0], right)
            b1 = rdma(BO.at[0], BIN.at[1], bag_s.at[1], bag_r.at[1], left)
            b2 = rdma(BOA.at[0], BIA.at[0], bag_s.at[2], bag_r.at[2], diag)
            b3 = rdma(BOB.at[0], BIB.at[0], bag_s.at[3], bag_r.at[3], diag)
            b0.start()
            b1.start()
            b2.start()
            b3.start()
            b0.wait_recv()
            dest[pl.ds(left * M_PER, M_PER), :] = BIN[0, :, :].astype(F32)
            b1.wait_recv()
            dest[pl.ds(right * M_PER, M_PER), :] = BIN[1, :, :].astype(F32)
            b2.wait_recv()
            dest[pl.ds(diag * M_PER, M_PER), :DH] = BIA[0, :, :].astype(F32)
            b3.wait_recv()
            dest[pl.ds(diag * M_PER, M_PER), DH:] = BIB[0, :, :].astype(F32)
            for r in (b0, b1, b2, b3):
                r.wait_send()

    return pl.pallas_call(
        body,
        out_shape=jax.ShapeDtypeStruct((N_DEV * M_PER, D), F32),
        in_specs=[pl.BlockSpec(memory_space=pltpu.VMEM)]
        + [pl.BlockSpec(memory_space=pltpu.MemorySpace.HBM)] * 6,
        out_specs=pl.BlockSpec(memory_space=pltpu.VMEM),
        scratch_shapes=[
            pltpu.VMEM((N_DEV * M_PER, D), F32),
            pltpu.VMEM((N_DEV * M_PER, D), F32),
            pltpu.VMEM((SLOTS, D, CH), F32),
            pltpu.VMEM((SLOTS, CH, D), F32),
            pltpu.VMEM((N_DEV, M_PER, DH), BF16),
            pltpu.VMEM((N_DEV, M_PER, DH), BF16),
            pltpu.VMEM((2, M_PER, D), BF16),
            pltpu.VMEM((2, M_PER, D), BF16),
            pltpu.VMEM((1, M_PER, DH), BF16),
            pltpu.VMEM((1, M_PER, DH), BF16),
            pltpu.VMEM((1, M_PER, DH), BF16),
            pltpu.VMEM((1, M_PER, DH), BF16),
            pltpu.VMEM((1, M_PER, D), BF16),
            pltpu.VMEM((1, M_PER, DH), BF16),
            pltpu.VMEM((1, M_PER, DH), BF16),
            pltpu.VMEM((2, M_PER, D), BF16),
            pltpu.VMEM((1, M_PER, DH), BF16),
            pltpu.VMEM((1, M_PER, DH), BF16),
        ]
        + [pltpu.SemaphoreType.DMA((N_DEV - 1,))] * 4
        + [pltpu.SemaphoreType.DMA((4,))] * 4
        + [
            pltpu.SemaphoreType.DMA((SLOTS,)),
            pltpu.SemaphoreType.DMA((SLOTS,)),
        ],
        compiler_params=pltpu.CompilerParams(
            collective_id=0,
            vmem_limit_bytes=60 * 1024 * 1024,
        ),
    )(x, Win0, Wout0, Win1, Wout1, Win2, Wout2)


# device time: 102360 ns/iter; 2.4815x vs baseline; 1.0127x over previous
import jax
import jax.numpy as jnp
from jax import lax
from jax.experimental import pallas as pl
from jax.experimental.pallas import tpu as pltpu

N_DEV = 4
M_PER = 64
D = 2048
DH = D // 2
H_PER = 4096
CH = 512
KS = H_PER // CH
N_LAYERS = 3
NCH = N_LAYERS * KS
SLOTS = 5

F32 = jnp.float32
BF16 = jnp.bfloat16


def kernel(x, Win0, Wout0, Win1, Wout1, Win2, Wout2):
    def body(
        x_ref, win0, wout0, win1, wout1, win2, wout2, out_ref,
        X, ACC, WINB, WOUTB,
        DRSO, DRSI, DOA, DOB, DIA, DIB,
        BO, BOA, BOB, BIN, BIA, BIB,
        drs_s, drs_r, bag_s, bag_r,
        wi_sem, wo_sem,
    ):
        my = lax.axis_index("i")
        left = (my - 1) % N_DEV
        right = (my + 1) % N_DEV
        diag = (my + 2) % N_DEV
        wins = [win0, win1, win2]
        wouts = [wout0, wout1, wout2]

        def w_copies(g):
            l, k = divmod(g, KS)
            s = g % SLOTS
            return (
                pltpu.make_async_copy(
                    wins[l].at[:, k * CH:(k + 1) * CH], WINB.at[s], wi_sem.at[s]
                ),
                pltpu.make_async_copy(
                    wouts[l].at[k * CH:(k + 1) * CH, :], WOUTB.at[s], wo_sem.at[s]
                ),
            )

        def rdma(src, dst, ssem, rsem, target):
            return pltpu.make_async_remote_copy(
                src_ref=src, dst_ref=dst, send_sem=ssem, recv_sem=rsem,
                device_id=(target,), device_id_type=pl.DeviceIdType.MESH,
            )

        for g in range(SLOTS):
            a, b = w_copies(g)
            a.start()
            b.start()

        barrier_sem = pltpu.get_barrier_semaphore()
        for nbr in [left, right, diag]:
            pl.semaphore_signal(
                barrier_sem, inc=1,
                device_id=(nbr,), device_id_type=pl.DeviceIdType.MESH,
            )
        pl.semaphore_wait(barrier_sem, 3)

        myrows = pl.ds(my * M_PER, M_PER)

        def broadcast(full, half_a, half_b, dest):
            BO[0, :, :] = full
            BOA[0, :, :] = half_a
            BOB[0, :, :] = half_b
            b2 = rdma(BOA.at[0], BIA.at[0], bag_s.at[2], bag_r.at[2], diag)
            b3 = rdma(BOB.at[0], BIB.at[0], bag_s.at[3], bag_r.at[3], diag)
            b0 = rdma(BO.at[0], BIN.at[0], bag_s.at[0], bag_r.at[0], right)
            b1 = rdma(BO.at[0], BIN.at[1], bag_s.at[1], bag_r.at[1], left)
            b2.start()
            b3.start()
            b0.start()
            b1.start()
            b0.wait_recv()
            dest[pl.ds(left * M_PER, M_PER), :] = BIN[0, :, :].astype(F32)
            b1.wait_recv()
            dest[pl.ds(right * M_PER, M_PER), :] = BIN[1, :, :].astype(F32)
            b2.wait_recv()
            dest[pl.ds(diag * M_PER, M_PER), :DH] = BIA[0, :, :].astype(F32)
            b3.wait_recv()
            dest[pl.ds(diag * M_PER, M_PER), DH:] = BIB[0, :, :].astype(F32)
            for r in (b0, b1, b2, b3):
                r.wait_send()

        X[myrows, :] = x_ref[...]
        broadcast(
            x_ref[...].astype(BF16),
            x_ref[:, :DH].astype(BF16),
            x_ref[:, DH:].astype(BF16),
            X,
        )

        for l in range(N_LAYERS):
            for k in range(KS):
                g = l * KS + k
                a, b = w_copies(g)
                a.wait()
                b.wait()
                s = g % SLOTS
                h = jnp.maximum(
                    jnp.dot(X[...], WINB[s], preferred_element_type=F32), 0.0
                )
                contrib = jnp.dot(h, WOUTB[s], preferred_element_type=F32)
                if k == 0:
                    ACC[...] = contrib
                else:
                    ACC[...] += contrib
                if g + SLOTS < NCH:
                    a2, b2 = w_copies(g + SLOTS)
                    a2.start()
                    b2.start()

            DOA[0, :, :] = ACC[pl.ds(diag * M_PER, M_PER), :DH].astype(BF16)
            DOB[0, :, :] = ACC[pl.ds(diag * M_PER, M_PER), DH:].astype(BF16)
            DRSO[0, :, :] = ACC[pl.ds(right * M_PER, M_PER), :].astype(BF16)
            DRSO[1, :, :] = ACC[pl.ds(left * M_PER, M_PER), :].astype(BF16)
            r2 = rdma(DOA.at[0], DIA.at[0], drs_s.at[2], drs_r.at[2], diag)
            r3 = rdma(DOB.at[0], DIB.at[0], drs_s.at[3], drs_r.at[3], diag)
            r0 = rdma(DRSO.at[0], DRSI.at[0], drs_s.at[0], drs_r.at[0], right)
            r1 = rdma(DRSO.at[1], DRSI.at[1], drs_s.at[1], drs_r.at[1], left)
            r2.start()
            r3.start()
            r0.start()
            r1.start()
            r0.wait_recv()
            r1.wait_recv()
            ACC[myrows, :] += DRSI[0, :, :].astype(F32) + DRSI[1, :, :].astype(F32)
            r2.wait_recv()
            ACC[myrows, :DH] += DIA[0, :, :].astype(F32)
            r3.wait_recv()
            ACC[myrows, DH:] += DIB[0, :, :].astype(F32)
            for r in (r0, r1, r2, r3):
                r.wait_send()

            dest = X if l < N_LAYERS - 1 else out_ref
            dest[myrows, :] = ACC[myrows, :]
            broadcast(
                ACC[myrows, :].astype(BF16),
                ACC[myrows, :DH].astype(BF16),
                ACC[myrows, DH:].astype(BF16),
                dest,
            )

    return pl.pallas_call(
        body,
        out_shape=jax.ShapeDtypeStruct((N_DEV * M_PER, D), F32),
        in_specs=[pl.BlockSpec(memory_space=pltpu.VMEM)]
        + [pl.BlockSpec(memory_space=pltpu.MemorySpace.HBM)] * 6,
        out_specs=pl.BlockSpec(memory_space=pltpu.VMEM),
        scratch_shapes=[
            pltpu.VMEM((N_DEV * M_PER, D), F32),
            pltpu.VMEM((N_DEV * M_PER, D), F32),
            pltpu.VMEM((SLOTS, D, CH), F32),
            pltpu.VMEM((SLOTS, CH, D), F32),
            pltpu.VMEM((2, M_PER, D), BF16),
            pltpu.VMEM((2, M_PER, D), BF16),
            pltpu.VMEM((1, M_PER, DH), BF16),
            pltpu.VMEM((1, M_PER, DH), BF16),
            pltpu.VMEM((1, M_PER, DH), BF16),
            pltpu.VMEM((1, M_PER, DH), BF16),
            pltpu.VMEM((1, M_PER, D), BF16),
            pltpu.VMEM((1, M_PER, DH), BF16),
            pltpu.VMEM((1, M_PER, DH), BF16),
            pltpu.VMEM((2, M_PER, D), BF16),
            pltpu.VMEM((1, M_PER, DH), BF16),
            pltpu.VMEM((1, M_PER, DH), BF16),
        ]
        + [pltpu.SemaphoreType.DMA((4,))] * 4
        + [
            pltpu.SemaphoreType.DMA((SLOTS,)),
            pltpu.SemaphoreType.DMA((SLOTS,)),
        ],
        compiler_params=pltpu.CompilerParams(
            collective_id=0,
            vmem_limit_bytes=60 * 1024 * 1024,
        ),
    )(x, Win0, Wout0, Win1, Wout1, Win2, Wout2)


# device time: 100394 ns/iter; 2.5301x vs baseline; 1.0196x over previous
import jax
import jax.numpy as jnp
from jax import lax
from jax.experimental import pallas as pl
from jax.experimental.pallas import tpu as pltpu

N_DEV = 4
M_PER = 64
D = 2048
DH = D // 2
H_PER = 4096
CH = 1024
KS = H_PER // CH
N_LAYERS = 3
NCH = N_LAYERS * KS
SLOTS = 3

F32 = jnp.float32
BF16 = jnp.bfloat16


def kernel(x, Win0, Wout0, Win1, Wout1, Win2, Wout2):
    def body(
        x_ref, win0, wout0, win1, wout1, win2, wout2, out_ref,
        X, ACC, WINB, WOUTB,
        DRSO, DRSI, DOA, DOB, DIA, DIB,
        BO, BOA, BOB, BIN, BIA, BIB,
        drs_s, drs_r, bag_s, bag_r,
        wi_sem, wo_sem,
    ):
        my = lax.axis_index("i")
        left = (my - 1) % N_DEV
        right = (my + 1) % N_DEV
        diag = (my + 2) % N_DEV
        wins = [win0, win1, win2]
        wouts = [wout0, wout1, wout2]

        def w_copies(g):
            l, k = divmod(g, KS)
            s = g % SLOTS
            return (
                pltpu.make_async_copy(
                    wins[l].at[:, k * CH:(k + 1) * CH], WINB.at[s], wi_sem.at[s]
                ),
                pltpu.make_async_copy(
                    wouts[l].at[k * CH:(k + 1) * CH, :], WOUTB.at[s], wo_sem.at[s]
                ),
            )

        def rdma(src, dst, ssem, rsem, target):
            return pltpu.make_async_remote_copy(
                src_ref=src, dst_ref=dst, send_sem=ssem, recv_sem=rsem,
                device_id=(target,), device_id_type=pl.DeviceIdType.MESH,
            )

        for g in range(SLOTS):
            a, b = w_copies(g)
            a.start()
            b.start()

        barrier_sem = pltpu.get_barrier_semaphore()
        for nbr in [left, right, diag]:
            pl.semaphore_signal(
                barrier_sem, inc=1,
                device_id=(nbr,), device_id_type=pl.DeviceIdType.MESH,
            )
        pl.semaphore_wait(barrier_sem, 3)

        myrows = pl.ds(my * M_PER, M_PER)

        def broadcast(full, half_a, half_b, dest):
            BO[0, :, :] = full
            BOA[0, :, :] = half_a
            BOB[0, :, :] = half_b
            b2 = rdma(BOA.at[0], BIA.at[0], bag_s.at[2], bag_r.at[2], diag)
            b3 = rdma(BOB.at[0], BIB.at[0], bag_s.at[3], bag_r.at[3], diag)
            b0 = rdma(BO.at[0], BIN.at[0], bag_s.at[0], bag_r.at[0], right)
            b1 = rdma(BO.at[0], BIN.at[1], bag_s.at[1], bag_r.at[1], left)
            b2.start()
            b3.start()
            b0.start()
            b1.start()
            b0.wait_recv()
            dest[pl.ds(left * M_PER, M_PER), :] = BIN[0, :, :].astype(F32)
            b1.wait_recv()
            dest[pl.ds(right * M_PER, M_PER), :] = BIN[1, :, :].astype(F32)
            b2.wait_recv()
            dest[pl.ds(diag * M_PER, M_PER), :DH] = BIA[0, :, :].astype(F32)
            b3.wait_recv()
            dest[pl.ds(diag * M_PER, M_PER), DH:] = BIB[0, :, :].astype(F32)
            for r in (b0, b1, b2, b3):
                r.wait_send()

        X[myrows, :] = x_ref[...]
        broadcast(
            x_ref[...].astype(BF16),
            x_ref[:, :DH].astype(BF16),
            x_ref[:, DH:].astype(BF16),
            X,
        )

        for l in range(N_LAYERS):
            for k in range(KS):
                g = l * KS + k
                a, b = w_copies(g)
                a.wait()
                b.wait()
                s = g % SLOTS
                h = jnp.maximum(
                    jnp.dot(X[...], WINB[s], preferred_element_type=F32), 0.0
                )
                contrib = jnp.dot(h, WOUTB[s], preferred_element_type=F32)
                if k == 0:
                    ACC[...] = contrib
                else:
                    ACC[...] += contrib
                if g + SLOTS < NCH and (g + SLOTS) // KS == l:
                    a2, b2 = w_copies(g + SLOTS)
                    a2.start()
                    b2.start()

            DOA[0, :, :] = ACC[pl.ds(diag * M_PER, M_PER), :DH].astype(BF16)
            DOB[0, :, :] = ACC[pl.ds(diag * M_PER, M_PER), DH:].astype(BF16)
            DRSO[0, :, :] = ACC[pl.ds(right * M_PER, M_PER), :].astype(BF16)
            DRSO[1, :, :] = ACC[pl.ds(left * M_PER, M_PER), :].astype(BF16)
            r2 = rdma(DOA.at[0], DIA.at[0], drs_s.at[2], drs_r.at[2], diag)
            r3 = rdma(DOB.at[0], DIB.at[0], drs_s.at[3], drs_r.at[3], diag)
            r0 = rdma(DRSO.at[0], DRSI.at[0], drs_s.at[0], drs_r.at[0], right)
            r1 = rdma(DRSO.at[1], DRSI.at[1], drs_s.at[1], drs_r.at[1], left)
            r2.start()
            r3.start()
            r0.start()
            r1.start()
            if l + 1 < N_LAYERS:
                for gnext in ((l + 1) * KS, (l + 1) * KS + 1):
                    a2, b2 = w_copies(gnext)
                    a2.start()
                    b2.start()
            r0.wait_recv()
            r1.wait_recv()
            ACC[myrows, :] += DRSI[0, :, :].astype(F32) + DRSI[1, :, :].astype(F32)
            r2.wait_recv()
            ACC[myrows, :DH] += DIA[0, :, :].astype(F32)
            r3.wait_recv()
            ACC[myrows, DH:] += DIB[0, :, :].astype(F32)
            for r in (r0, r1, r2, r3):
                r.wait_send()

            dest = X if l < N_LAYERS - 1 else out_ref
            if l + 1 < N_LAYERS:
                a2, b2 = w_copies((l + 1) * KS + 2)
                a2.start()
                b2.start()
            dest[myrows, :] = ACC[myrows, :]
            broadcast(
                ACC[myrows, :].astype(BF16),
                ACC[myrows, :DH].astype(BF16),
                ACC[myrows, DH:].astype(BF16),
                dest,
            )

    return pl.pallas_call(
        body,
        out_shape=jax.ShapeDtypeStruct((N_DEV * M_PER, D), F32),
        in_specs=[pl.BlockSpec(memory_space=pltpu.VMEM)]
        + [pl.BlockSpec(memory_space=pltpu.MemorySpace.HBM)] * 6,
        out_specs=pl.BlockSpec(memory_space=pltpu.VMEM),
        scratch_shapes=[
            pltpu.VMEM((N_DEV * M_PER, D), F32),
            pltpu.VMEM((N_DEV * M_PER, D), F32),
            pltpu.VMEM((SLOTS, D, CH), F32),
            pltpu.VMEM((SLOTS, CH, D), F32),
            pltpu.VMEM((2, M_PER, D), BF16),
            pltpu.VMEM((2, M_PER, D), BF16),
            pltpu.VMEM((1, M_PER, DH), BF16),
            pltpu.VMEM((1, M_PER, DH), BF16),
            pltpu.VMEM((1, M_PER, DH), BF16),
            pltpu.VMEM((1, M_PER, DH), BF16),
            pltpu.VMEM((1, M_PER, D), BF16),
            pltpu.VMEM((1, M_PER, DH), BF16),
            pltpu.VMEM((1, M_PER, DH), BF16),
            pltpu.VMEM((2, M_PER, D), BF16),
            pltpu.VMEM((1, M_PER, DH), BF16),
            pltpu.VMEM((1, M_PER, DH), BF16),
        ]
        + [pltpu.SemaphoreType.DMA((4,))] * 4
        + [
            pltpu.SemaphoreType.DMA((SLOTS,)),
            pltpu.SemaphoreType.DMA((SLOTS,)),
        ],
        compiler_params=pltpu.CompilerParams(
            collective_id=0,
            vmem_limit_bytes=60 * 1024 * 1024,
        ),
    )(x, Win0, Wout0, Win1, Wout1, Win2, Wout2)
